# baseline (speedup 1.0000x reference)
import dataclasses
import functools

import jax
import jax.numpy as jnp
from jax import lax
from jax.experimental import pallas as pl
from jax.experimental.pallas import tpu as pltpu
from jax.experimental.pallas import tpu_sc as plsc

NW = 32
CHUNK_ROWS = 4


@functools.partial(jax.jit, static_argnums=(4, 5))
def _sc_forward(idx, emb_table, i_emb_table, g_emb_table, B, F):
    D = emb_table.shape[1]
    rows_per_w = B // NW
    chunk_idx = CHUNK_ROWS * F
    nchunk = rows_per_w // CHUNK_ROWS

    mesh = plsc.VectorSubcoreMesh(core_axis_name="c", subcore_axis_name="s")

    cp = pltpu.CompilerParams()
    if "needs_layout_passes" in pltpu.CompilerParams.__dataclass_fields__:
        cp = dataclasses.replace(cp, needs_layout_passes=False)
    if "use_tc_tiling_on_sc" in pltpu.CompilerParams.__dataclass_fields__:
        cp = dataclasses.replace(cp, use_tc_tiling_on_sc=False)

    groups_per_w = rows_per_w // D

    @functools.partial(
        pl.kernel,
        out_type=jax.ShapeDtypeStruct((B // D, D), jnp.float32),
        mesh=mesh,
        compiler_params=cp,
        scratch_types=[
            pltpu.VMEM((nchunk, chunk_idx), jnp.int32),
            pltpu.VMEM((chunk_idx, D), jnp.float32),
            pltpu.VMEM((chunk_idx, D), jnp.float32),
            pltpu.VMEM((chunk_idx, D), jnp.float32),
            pltpu.VMEM((chunk_idx, D), jnp.float32),
            pltpu.VMEM((chunk_idx, D), jnp.float32),
            pltpu.VMEM((chunk_idx, D), jnp.float32),
            pltpu.VMEM((groups_per_w, D), jnp.float32),
            pltpu.SemaphoreType.DMA,
            pltpu.SemaphoreType.DMA,
            pltpu.SemaphoreType.DMA,
            pltpu.SemaphoreType.DMA,
            pltpu.SemaphoreType.DMA,
            pltpu.SemaphoreType.DMA,
        ],
    )
    def sc_kernel(idx_hbm, e_hbm, i_hbm, g_hbm, out_hbm,
                  idx_v, e0, i0, g0, e1, i1, g1, out_v,
                  se0, si0, sg0, se1, si1, sg1):
        cid = lax.axis_index("c")
        sid = lax.axis_index("s")
        wid = sid * 2 + cid

        pltpu.sync_copy(idx_hbm.at[wid], idx_v)

        def issue(j, eb, ib, gb, se, si, sg):
            row = idx_v.at[j]
            pltpu.async_copy(e_hbm.at[row], eb, se)
            pltpu.async_copy(i_hbm.at[row], ib, si)
            pltpu.async_copy(g_hbm.at[row], gb, sg)

        def wait(j, eb, ib, gb, se, si, sg):
            row = idx_v.at[j]
            pltpu.make_async_copy(e_hbm.at[row], eb, se).wait()
            pltpu.make_async_copy(i_hbm.at[row], ib, si).wait()
            pltpu.make_async_copy(g_hbm.at[row], gb, sg).wait()

        chunks_per_group = D // CHUNK_ROWS
        lanes = lax.iota(jnp.int32, 16)

        def compute(j, eb, ib, gb):
            g = j // chunks_per_group
            q = j % chunks_per_group
            vec = out_v[g]
            for r in range(CHUNK_ROWS):
                acc = eb[r * F] * ib[r * F] * gb[r * F]
                for f in range(1, F):
                    k = r * F + f
                    acc = acc + eb[k] * ib[k] * gb[k]
                s = jnp.sum(acc)
                vec = jnp.where(lanes == q * CHUNK_ROWS + r, s, vec)
            out_v[g] = vec

        issue(0, e0, i0, g0, se0, si0, sg0)

        @pl.loop(0, nchunk, step=2)
        def _(j):
            issue(j + 1, e1, i1, g1, se1, si1, sg1)
            wait(j, e0, i0, g0, se0, si0, sg0)
            compute(j, e0, i0, g0)

            @pl.when(j + 2 < nchunk)
            def _():
                issue(j + 2, e0, i0, g0, se0, si0, sg0)

            wait(j + 1, e1, i1, g1, se1, si1, sg1)
            compute(j + 1, e1, i1, g1)

        pltpu.sync_copy(out_v, out_hbm.at[pl.ds(wid * groups_per_w, groups_per_w)])

    return sc_kernel(idx, emb_table, i_emb_table, g_emb_table)


def kernel(sparse_input, emb_table, i_emb_table, g_emb_table):
    B, F = sparse_input.shape
    rows_per_w = B // NW
    chunk_idx = CHUNK_ROWS * F
    nchunk = rows_per_w // CHUNK_ROWS
    idx = sparse_input.astype(jnp.int32).reshape(NW, nchunk, chunk_idx)
    out = _sc_forward(idx, emb_table, i_emb_table, g_emb_table, B, F)
    return out.reshape(B)
